# baseline (speedup 1.0000x reference)
import jax
import jax.numpy as jnp
from jax.experimental import pallas as pl

N = 50000
G = 8
D = 64
H = 4
OC = D // H
DFF = 256
L = 2
NUM_RB = 4


def _layer_norm(x, g, b):
    m = x.mean(axis=-1, keepdims=True)
    v = ((x - m) ** 2).mean(axis=-1, keepdims=True)
    return (x - m) / jnp.sqrt(v + 1e-5) * g + b


def _transformer_conv(x, edge_attr, edge_index, Wq, bq, Wk, bk, Wv, bv, We, Ws, bs):
    n = x.shape[0]
    src = edge_index[0]
    dst = edge_index[1]
    q = (x @ Wq + bq).reshape(n, H, OC)
    k = (x @ Wk + bk).reshape(n, H, OC)
    v = (x @ Wv + bv).reshape(n, H, OC)
    e = (edge_attr @ We).reshape(-1, H, OC)
    k_e = k[src] + e
    alpha = (q[dst] * k_e).sum(-1) / jnp.sqrt(float(OC))
    amax = jax.ops.segment_max(alpha, dst, num_segments=n)
    amax = jnp.where(jnp.isfinite(amax), amax, 0.0)
    ex = jnp.exp(alpha - amax[dst])
    denom = jax.ops.segment_sum(ex, dst, num_segments=n)
    attn = ex / (denom[dst] + 1e-16)
    msg = (v[src] + e) * attn[:, :, None]
    out = jax.ops.segment_sum(msg, dst, num_segments=n).reshape(n, H * OC)
    return out + x @ Ws + bs


def _actor_body(x_ref, w_ref, b_ref, o_ref):
    o_ref[...] = x_ref[...] @ w_ref[...] + b_ref[...]


def kernel(power_alloc, beam_alloc, node_power_attn, edge_power_attn, edge_index, ptr, batch, params):
    p = params
    n = power_alloc.shape[0]
    resource = jnp.concatenate([power_alloc, beam_alloc], axis=2).reshape(n, -1)
    inp = resource @ p["W_in"] + p["b_in"]
    x = node_power_attn.reshape(n, -1) @ p["W_emb"] + p["b_emb"]
    edge_attr = edge_power_attn.reshape(edge_power_attn.shape[0], -1)
    for l in range(L):
        x = x + inp
        x2 = _transformer_conv(x, edge_attr, edge_index, p["Wq"][l], p["bq"][l], p["Wk"][l], p["bk"][l], p["Wv"][l], p["bv"][l], p["We"][l], p["Ws"][l], p["bs"][l])
        x = _layer_norm(x + x2, p["g1"][l], p["be1"][l])
        x2 = jax.nn.relu(x @ p["W1"][l] + p["b1"][l]) @ p["W2"][l] + p["b2"][l]
        x = _layer_norm(x + x2, p["g2"][l], p["be2"][l])
    ones = jnp.ones((n, 1), dtype=x.dtype)
    sums = jax.ops.segment_sum(x, batch, num_segments=G)
    cnts = jax.ops.segment_sum(ones, batch, num_segments=G)
    pooled = sums / jnp.maximum(cnts, 1.0)
    value = (pooled @ p["W_critic"] + p["b_critic"])[:, 0]
    link_rb_logit = pl.pallas_call(
        _actor_body,
        out_shape=jax.ShapeDtypeStruct((n, NUM_RB), jnp.float32),
        grid=(10,),
        in_specs=[
            pl.BlockSpec((n // 10, D), lambda i: (i, 0)),
            pl.BlockSpec((D, NUM_RB), lambda i: (0, 0)),
            pl.BlockSpec((NUM_RB,), lambda i: (0,)),
        ],
        out_specs=pl.BlockSpec((n // 10, NUM_RB), lambda i: (i, 0)),
    )(x, p["W_actor"], p["b_actor"])
    return x, value, link_rb_logit


# speedup vs baseline: 16.5836x; 16.5836x over previous
import functools

import jax
import jax.numpy as jnp
from jax import lax
from jax.experimental import pallas as pl
from jax.experimental.pallas import tpu as pltpu
from jax.experimental.pallas import tpu_sc as plsc

N = 50000
E = 800000
G = 8
D = 64
H = 4
OC = D // H
DFF = 256
L = 2
NUM_RB = 4

NC = 2
NS = 16
NW = NC * NS

EPAD = 819200
GB = 1024
PW_G = EPAD // NW
NCH_G = PW_G // GB
PW_S = EPAD // NS
NCH_S = PW_S // GB

MC = 72
NWIN = 8448
NROUND = 3
ACCR = 8576
ZROWS = ACCR // NS

BN = 2000
BE = 2048


def _prologue_body(pa_ref, ba_ref, npa_ref, wpa_ref, wba_ref, bin_ref,
                   wemb_ref, bemb_ref, inp_ref, x0_ref):
    inp_ref[...] = (pa_ref[...] @ wpa_ref[...] + ba_ref[...] @ wba_ref[...]
                    + bin_ref[...])
    x0_ref[...] = npa_ref[...] @ wemb_ref[...] + bemb_ref[...]


def _qkv_body(x_ref, inp_ref, wq_ref, bq_ref, wk_ref, bk_ref, wv_ref, bv_ref,
              xi_ref, q_ref, k_ref, v_ref):
    xi = x_ref[...] + inp_ref[...]
    xi_ref[...] = xi
    q_ref[...] = xi @ wq_ref[...] + bq_ref[...]
    k_ref[...] = xi @ wk_ref[...] + bk_ref[...]
    v_ref[...] = xi @ wv_ref[...] + bv_ref[...]


def _edge_body(qd_ref, ks_ref, vs_ref, a_ref, we_ref, o_ref):
    e = a_ref[...] @ we_ref[...]
    qd = qd_ref[...]
    ke = ks_ref[...] + e
    ve = vs_ref[...] + e
    parts = []
    exs = []
    for h in range(H):
        sl = slice(h * OC, (h + 1) * OC)
        alpha = jnp.sum(qd[:, sl] * ke[:, sl], axis=1, keepdims=True) * 0.25
        ex = jnp.exp(alpha)
        parts.append(ve[:, sl] * ex)
        exs.append(ex)
    z = jnp.zeros_like(exs[0])
    o_ref[...] = jnp.concatenate(parts + exs + [z, z, z, z], axis=1)


def _layer_norm(x, g, b):
    m = jnp.mean(x, axis=-1, keepdims=True)
    v = jnp.mean((x - m) ** 2, axis=-1, keepdims=True)
    return (x - m) * jax.lax.rsqrt(v + 1e-5) * g + b


def _node_body(xi_ref, acc_ref, ws_ref, bs_ref, g1_ref, be1_ref, w1_ref,
               b1_ref, w2_ref, b2_ref, g2_ref, be2_ref, wa_ref, ba_ref,
               xn_ref, logit_ref):
    acc = acc_ref[...]
    aggs = []
    for h in range(H):
        den = acc[:, 64 + h:65 + h] + 1e-16
        aggs.append(acc[:, h * OC:(h + 1) * OC] / den)
    agg = jnp.concatenate(aggs, axis=1)
    xi = xi_ref[...]
    x2 = agg + xi @ ws_ref[...] + bs_ref[...]
    y = _layer_norm(xi + x2, g1_ref[...], be1_ref[...])
    z = jax.nn.relu(y @ w1_ref[...] + b1_ref[...]) @ w2_ref[...] + b2_ref[...]
    xn = _layer_norm(y + z, g2_ref[...], be2_ref[...])
    xn_ref[...] = xn
    logit_ref[...] = xn @ wa_ref[...] + ba_ref[...]


def _pool_body(x_ref, b_ref, wc_ref, bc_ref, sums_ref, cnts_ref, val_ref):
    i = pl.program_id(0)

    @pl.when(i == 0)
    def _init():
        sums_ref[...] = jnp.zeros_like(sums_ref)
        cnts_ref[...] = jnp.zeros_like(cnts_ref)

    x = x_ref[...]
    bv = b_ref[...]
    rows_s = []
    rows_c = []
    for g in range(G):
        mask = (bv == g).astype(jnp.float32)
        rows_s.append(jnp.sum(mask * x, axis=0, keepdims=True))
        rows_c.append(jnp.sum(mask, axis=0, keepdims=True) *
                      jnp.ones((1, D), jnp.float32))
    sums_ref[...] += jnp.concatenate(rows_s, axis=0)
    cnts_ref[...] += jnp.concatenate(rows_c, axis=0)

    @pl.when(i == pl.num_programs(0) - 1)
    def _fin():
        pooled = sums_ref[...] / jnp.maximum(cnts_ref[...], 1.0)
        val_ref[...] = pooled @ wc_ref[...] + bc_ref[...]


_MESH = plsc.VectorSubcoreMesh(core_axis_name="c", subcore_axis_name="s",
                               num_cores=NC, num_subcores=NS)


@functools.partial(
    pl.kernel,
    out_type=[jax.ShapeDtypeStruct((EPAD, D), jnp.float32)] * 3,
    mesh=_MESH,
    compiler_params=pltpu.CompilerParams(use_tc_tiling_on_sc=False),
    scratch_types=[
        pltpu.VMEM((8, 128), jnp.int32),
        pltpu.VMEM((GB, D), jnp.float32),
        pltpu.SemaphoreType.DMA,
    ],
)
def _sc_gather(src2_hbm, dst2_hbm, q_hbm, k_hbm, v_hbm,
               qd_hbm, ks_hbm, vs_hbm, idx_v, rows_v, sem):
    c = lax.axis_index("c")
    s = lax.axis_index("s")
    wid = c * NS + s
    base = wid * PW_G

    def table_gather(tab_hbm, out_hbm, off, row0):
        descs = [pltpu.async_copy(tab_hbm.at[idx_v.at[j]],
                                  rows_v.at[pl.ds(j * 128, 128)], sem)
                 for j in range(8)]
        for d in descs:
            d.wait()
        pltpu.sync_copy(rows_v, out_hbm.at[pl.ds(off, GB)])

    def chunk(i, _):
        off = pl.multiple_of(base + i * GB, GB)
        row0 = pl.multiple_of(off // 128, 8)
        pltpu.sync_copy(src2_hbm.at[pl.ds(row0, 8)], idx_v)
        table_gather(k_hbm, ks_hbm, off, row0)
        table_gather(v_hbm, vs_hbm, off, row0)
        pltpu.sync_copy(dst2_hbm.at[pl.ds(row0, 8)], idx_v)
        table_gather(q_hbm, qd_hbm, off, row0)
        return 0

    lax.fori_loop(0, NCH_G, chunk, 0)


@functools.partial(
    pl.kernel,
    out_type=jax.ShapeDtypeStruct((2 * NROUND * ACCR, MC), jnp.float32),
    mesh=_MESH,
    compiler_params=pltpu.CompilerParams(use_tc_tiling_on_sc=False),
    scratch_types=[
        pltpu.VMEM((8, 128), jnp.int32),
        pltpu.VMEM((8, 128), jnp.int32),
        pltpu.VMEM((GB, MC), jnp.float32),
        pltpu.VMEM_SHARED((ACCR, MC), jnp.float32),
    ],
)
def _sc_scatter(dst2_hbm, msg_hbm, zero_hbm, out_hbm,
                didx_v, lidx_v, rows_v, acc_sh):
    c = lax.axis_index("c")
    s = lax.axis_index("s")
    lanes = lax.iota(jnp.int32, 16)

    def do_round(r, _):
        w = 2 * r + c
        nbase = w * NWIN

        pltpu.sync_copy(zero_hbm, acc_sh.at[pl.ds(s * ZROWS, ZROWS)])
        plsc.subcore_barrier()

        def chunk(i, _):
            off = pl.multiple_of(s * PW_S + i * GB, GB)
            row0 = pl.multiple_of(off // 128, 8)
            pltpu.sync_copy(dst2_hbm.at[pl.ds(row0, 8)], didx_v)
            pltpu.sync_copy(msg_hbm.at[pl.ds(off, GB)], rows_v)
            for j in range(8):
                for t in range(8):
                    dvec = didx_v[j, pl.ds(t * 16, 16)]
                    lvec = dvec - nbase
                    gpos = off + (j * 128 + t * 16) + lanes
                    ok = (lvec >= 0) & (lvec < NWIN) & (gpos < E)
                    trash = NWIN + (dvec & 127)
                    lvec = jnp.where(ok, lvec, trash)
                    lidx_v[j, pl.ds(t * 16, 16)] = lvec
            for j in range(8):
                pltpu.sync_copy(rows_v.at[pl.ds(j * 128, 128)],
                                acc_sh.at[lidx_v.at[j]], add=True)
            return 0

        lax.fori_loop(0, NCH_S, chunk, 0)

        plsc.subcore_barrier()
        pltpu.sync_copy(acc_sh.at[pl.ds(s * ZROWS, ZROWS)],
                        out_hbm.at[pl.ds(w * ACCR + s * ZROWS, ZROWS)])
        plsc.subcore_barrier()
        return 0

    lax.fori_loop(0, NROUND, do_round, 0)


def _tc_call(body, grid, in_specs, out_specs, out_shape, args):
    return pl.pallas_call(
        body,
        grid=grid,
        in_specs=in_specs,
        out_specs=out_specs,
        out_shape=out_shape,
    )(*args)


def _row(x):
    return x.reshape(1, -1)


def kernel(power_alloc, beam_alloc, node_power_attn, edge_power_attn,
           edge_index, ptr, batch, params):
    p = params
    f32 = jnp.float32

    pa = power_alloc.reshape(N, 16)
    ba = beam_alloc.reshape(N, 8)
    npa = node_power_attn.reshape(N, -1)
    ea = edge_power_attn.reshape(E, -1)
    ea_p = jnp.concatenate([ea, jnp.zeros((EPAD - E, 16), f32)], axis=0)
    src_p = jnp.concatenate([edge_index[0], jnp.zeros((EPAD - E,), jnp.int32)])
    dst_p = jnp.concatenate([edge_index[1], jnp.zeros((EPAD - E,), jnp.int32)])
    src2 = src_p.reshape(EPAD // 128, 128)
    dst2 = dst_p.reshape(EPAD // 128, 128)
    batch2 = batch.reshape(N, 1)
    zero_rows = jnp.zeros((ZROWS, MC), f32)

    w_in = p["W_in"]
    idx_pa = [6 * rb + j for rb in range(NUM_RB) for j in range(4)]
    idx_ba = [6 * rb + 4 + j for rb in range(NUM_RB) for j in range(2)]
    wpa = w_in[jnp.array(idx_pa)]
    wba = w_in[jnp.array(idx_ba)]

    nb = N // BN
    spec_n64 = pl.BlockSpec((BN, D), lambda i: (i, 0))
    spec_w = lambda r, c: pl.BlockSpec((r, c), lambda i: (0, 0))

    inp, x = _tc_call(
        _prologue_body, (nb,),
        [pl.BlockSpec((BN, 16), lambda i: (i, 0)),
         pl.BlockSpec((BN, 8), lambda i: (i, 0)),
         pl.BlockSpec((BN, 16), lambda i: (i, 0)),
         spec_w(16, D), spec_w(8, D), spec_w(1, D),
         spec_w(16, D), spec_w(1, D)],
        [spec_n64, spec_n64],
        [jax.ShapeDtypeStruct((N, D), f32)] * 2,
        (pa, ba, npa, wpa, wba, _row(p["b_in"]), p["W_emb"], _row(p["b_emb"])),
    )

    logit = None
    for l in range(L):
        xi, q_t, k_t, v_t = _tc_call(
            _qkv_body, (nb,),
            [spec_n64, spec_n64,
             spec_w(D, D), spec_w(1, D), spec_w(D, D), spec_w(1, D),
             spec_w(D, D), spec_w(1, D)],
            [spec_n64] * 4,
            [jax.ShapeDtypeStruct((N, D), f32)] * 4,
            (x, inp, p["Wq"][l], _row(p["bq"][l]), p["Wk"][l],
             _row(p["bk"][l]), p["Wv"][l], _row(p["bv"][l])),
        )

        qd, ks, vs = _sc_gather(src2, dst2, q_t, k_t, v_t)

        neb = EPAD // BE
        spec_e64 = pl.BlockSpec((BE, D), lambda i: (i, 0))
        msg = _tc_call(
            _edge_body, (neb,),
            [spec_e64, spec_e64, spec_e64,
             pl.BlockSpec((BE, 16), lambda i: (i, 0)),
             pl.BlockSpec((16, D), lambda i: (0, 0))],
            pl.BlockSpec((BE, MC), lambda i: (i, 0)),
            jax.ShapeDtypeStruct((EPAD, MC), f32),
            (qd, ks, vs, ea_p, p["We"][l]),
        )

        acc_pad = _sc_scatter(dst2, msg, zero_rows)
        acc = jnp.concatenate(
            [acc_pad[w * ACCR:w * ACCR + NWIN] for w in range(2 * NROUND)],
            axis=0)[:N]

        x, logit = _tc_call(
            _node_body, (nb,),
            [spec_n64, pl.BlockSpec((BN, MC), lambda i: (i, 0)),
             spec_w(D, D), spec_w(1, D), spec_w(1, D), spec_w(1, D),
             spec_w(D, DFF), spec_w(1, DFF), spec_w(DFF, D), spec_w(1, D),
             spec_w(1, D), spec_w(1, D), spec_w(D, NUM_RB), spec_w(1, NUM_RB)],
            [spec_n64, pl.BlockSpec((BN, NUM_RB), lambda i: (i, 0))],
            [jax.ShapeDtypeStruct((N, D), f32),
             jax.ShapeDtypeStruct((N, NUM_RB), f32)],
            (xi, acc, p["Ws"][l], _row(p["bs"][l]), _row(p["g1"][l]),
             _row(p["be1"][l]), p["W1"][l], _row(p["b1"][l]), p["W2"][l],
             _row(p["b2"][l]), _row(p["g2"][l]), _row(p["be2"][l]),
             p["W_actor"], _row(p["b_actor"])),
        )

    sums, cnts, val = _tc_call(
        _pool_body, (nb,),
        [spec_n64, pl.BlockSpec((BN, 1), lambda i: (i, 0)),
         spec_w(D, 1), spec_w(1, 1)],
        [pl.BlockSpec((G, D), lambda i: (0, 0)),
         pl.BlockSpec((G, D), lambda i: (0, 0)),
         pl.BlockSpec((G, 1), lambda i: (0, 0))],
        [jax.ShapeDtypeStruct((G, D), f32), jax.ShapeDtypeStruct((G, D), f32),
         jax.ShapeDtypeStruct((G, 1), f32)],
        (x, batch2, p["W_critic"], p["b_critic"].reshape(1, 1)),
    )
    value = val[:, 0]
    return x, value, logit


# speedup vs baseline: 18.7568x; 1.1310x over previous
import functools

import jax
import jax.numpy as jnp
from jax import lax
from jax.experimental import pallas as pl
from jax.experimental.pallas import tpu as pltpu
from jax.experimental.pallas import tpu_sc as plsc

N = 50000
E = 800000
G = 8
D = 64
H = 4
OC = D // H
DFF = 256
L = 2
NUM_RB = 4

NC = 2
NS = 16
NW = NC * NS

EPAD = 819200
GB = 1024
PW_G = EPAD // NW
GBG = 256
NCH2 = PW_G // GBG // 2
PW_S = EPAD // NS
NCH_S = PW_S // GB

MC = 72
NWIN = 8448
NROUND = 3
ACCR = 8576
ZROWS = ACCR // NS

BN = 2000
BE = 2048


def _prologue_body(pa_ref, ba_ref, npa_ref, wpa_ref, wba_ref, bin_ref,
                   wemb_ref, bemb_ref, inp_ref, x0_ref):
    inp_ref[...] = (pa_ref[...] @ wpa_ref[...] + ba_ref[...] @ wba_ref[...]
                    + bin_ref[...])
    x0_ref[...] = npa_ref[...] @ wemb_ref[...] + bemb_ref[...]


def _qkv_body(x_ref, inp_ref, wq_ref, bq_ref, wk_ref, bk_ref, wv_ref, bv_ref,
              xi_ref, q_ref, kv_ref):
    xi = x_ref[...] + inp_ref[...]
    xi_ref[...] = xi
    q_ref[...] = xi @ wq_ref[...] + bq_ref[...]
    kv_ref[...] = jnp.concatenate(
        [xi @ wk_ref[...] + bk_ref[...], xi @ wv_ref[...] + bv_ref[...]],
        axis=1)


def _edge_body(qd_ref, kvs_ref, a_ref, we_ref, o_ref):
    e = a_ref[...] @ we_ref[...]
    qd = qd_ref[...]
    kvs = kvs_ref[...]
    ke = kvs[:, :D] + e
    ve = kvs[:, D:] + e
    parts = []
    exs = []
    for h in range(H):
        sl = slice(h * OC, (h + 1) * OC)
        alpha = jnp.sum(qd[:, sl] * ke[:, sl], axis=1, keepdims=True) * 0.25
        ex = jnp.exp(alpha)
        parts.append(ve[:, sl] * ex)
        exs.append(ex)
    z = jnp.zeros_like(exs[0])
    o_ref[...] = jnp.concatenate(parts + exs + [z, z, z, z], axis=1)


def _layer_norm(x, g, b):
    m = jnp.mean(x, axis=-1, keepdims=True)
    v = jnp.mean((x - m) ** 2, axis=-1, keepdims=True)
    return (x - m) * jax.lax.rsqrt(v + 1e-5) * g + b


def _node_body(xi_ref, acc_ref, ws_ref, bs_ref, g1_ref, be1_ref, w1_ref,
               b1_ref, w2_ref, b2_ref, g2_ref, be2_ref, wa_ref, ba_ref,
               xn_ref, logit_ref):
    acc = acc_ref[...]
    aggs = []
    for h in range(H):
        den = acc[:, 64 + h:65 + h] + 1e-16
        aggs.append(acc[:, h * OC:(h + 1) * OC] / den)
    agg = jnp.concatenate(aggs, axis=1)
    xi = xi_ref[...]
    x2 = agg + xi @ ws_ref[...] + bs_ref[...]
    y = _layer_norm(xi + x2, g1_ref[...], be1_ref[...])
    z = jax.nn.relu(y @ w1_ref[...] + b1_ref[...]) @ w2_ref[...] + b2_ref[...]
    xn = _layer_norm(y + z, g2_ref[...], be2_ref[...])
    xn_ref[...] = xn
    logit_ref[...] = xn @ wa_ref[...] + ba_ref[...]


def _pool_body(x_ref, b_ref, wc_ref, bc_ref, sums_ref, cnts_ref, val_ref):
    i = pl.program_id(0)

    @pl.when(i == 0)
    def _init():
        sums_ref[...] = jnp.zeros_like(sums_ref)
        cnts_ref[...] = jnp.zeros_like(cnts_ref)

    x = x_ref[...]
    bv = b_ref[...]
    rows_s = []
    rows_c = []
    for g in range(G):
        mask = (bv == g).astype(jnp.float32)
        rows_s.append(jnp.sum(mask * x, axis=0, keepdims=True))
        rows_c.append(jnp.sum(mask, axis=0, keepdims=True) *
                      jnp.ones((1, D), jnp.float32))
    sums_ref[...] += jnp.concatenate(rows_s, axis=0)
    cnts_ref[...] += jnp.concatenate(rows_c, axis=0)

    @pl.when(i == pl.num_programs(0) - 1)
    def _fin():
        pooled = sums_ref[...] / jnp.maximum(cnts_ref[...], 1.0)
        val_ref[...] = pooled @ wc_ref[...] + bc_ref[...]


_MESH = plsc.VectorSubcoreMesh(core_axis_name="c", subcore_axis_name="s",
                               num_cores=NC, num_subcores=NS)


@functools.partial(
    pl.kernel,
    out_type=[jax.ShapeDtypeStruct((EPAD, 2 * D), jnp.float32),
              jax.ShapeDtypeStruct((EPAD, D), jnp.float32)],
    mesh=_MESH,
    compiler_params=pltpu.CompilerParams(use_tc_tiling_on_sc=False),
    scratch_types=[
        pltpu.VMEM((4, 128), jnp.int32),
        pltpu.VMEM((4, 128), jnp.int32),
        pltpu.VMEM((GBG, 2 * D), jnp.float32),
        pltpu.VMEM((GBG, 2 * D), jnp.float32),
        pltpu.VMEM((GBG, D), jnp.float32),
        pltpu.VMEM((GBG, D), jnp.float32),
        pltpu.SemaphoreType.DMA,
        pltpu.SemaphoreType.DMA,
        pltpu.SemaphoreType.DMA,
        pltpu.SemaphoreType.DMA,
        pltpu.SemaphoreType.DMA,
        pltpu.SemaphoreType.DMA,
    ],
)
def _sc_gather(sd_hbm, kv_hbm, q_hbm, kvs_hbm, qd_hbm,
               ib0, ib1, kvb0, kvb1, qdb0, qdb1,
               isem0, isem1, gsem0, gsem1, wsem0, wsem1):
    c = lax.axis_index("c")
    s = lax.axis_index("s")
    wid = c * NS + s
    cbase = wid * (2 * NCH2)

    bufs = ((ib0, kvb0, qdb0, isem0, gsem0, wsem0),
            (ib1, kvb1, qdb1, isem1, gsem1, wsem1))

    for half, (ib, kvb, qdb, isem, gsem, wsem) in enumerate(bufs):
        r0 = pl.multiple_of((cbase + half) * 4, 4)
        pltpu.async_copy(sd_hbm.at[pl.ds(r0, 4)], ib, isem)

    def body(i2, _):
        for half, (ib, kvb, qdb, isem, gsem, wsem) in enumerate(bufs):
            ci = cbase + 2 * i2 + half
            off = pl.multiple_of(ci * GBG, GBG)
            r0 = pl.multiple_of(ci * 4, 4)
            pltpu.make_async_copy(sd_hbm.at[pl.ds(r0, 4)], ib, isem).wait()

            @pl.when(i2 > 0)
            def _wait_writes():
                poff = pl.multiple_of(off - 2 * GBG, GBG)
                pltpu.make_async_copy(
                    kvb, kvs_hbm.at[pl.ds(poff, GBG)], wsem).wait()
                pltpu.make_async_copy(
                    qdb, qd_hbm.at[pl.ds(poff, GBG)], wsem).wait()

            g = [pltpu.async_copy(kv_hbm.at[ib.at[j]],
                                  kvb.at[pl.ds(j * 128, 128)], gsem)
                 for j in range(2)]
            g += [pltpu.async_copy(q_hbm.at[ib.at[2 + j]],
                                   qdb.at[pl.ds(j * 128, 128)], gsem)
                  for j in range(2)]
            for d in g:
                d.wait()

            @pl.when(i2 < NCH2 - 1)
            def _prefetch():
                r2 = pl.multiple_of((ci + 2) * 4, 4)
                pltpu.async_copy(sd_hbm.at[pl.ds(r2, 4)], ib, isem)

            pltpu.async_copy(kvb, kvs_hbm.at[pl.ds(off, GBG)], wsem)
            pltpu.async_copy(qdb, qd_hbm.at[pl.ds(off, GBG)], wsem)
        return 0

    lax.fori_loop(0, NCH2, body, 0)

    for half, (ib, kvb, qdb, isem, gsem, wsem) in enumerate(bufs):
        loff = pl.multiple_of((cbase + 2 * (NCH2 - 1) + half) * GBG, GBG)
        pltpu.make_async_copy(kvb, kvs_hbm.at[pl.ds(loff, GBG)], wsem).wait()
        pltpu.make_async_copy(qdb, qd_hbm.at[pl.ds(loff, GBG)], wsem).wait()


@functools.partial(
    pl.kernel,
    out_type=jax.ShapeDtypeStruct((2 * NROUND * ACCR, MC), jnp.float32),
    mesh=_MESH,
    compiler_params=pltpu.CompilerParams(use_tc_tiling_on_sc=False),
    scratch_types=[
        pltpu.VMEM((8, 128), jnp.int32),
        pltpu.VMEM((8, 128), jnp.int32),
        pltpu.VMEM((GB, MC), jnp.float32),
        pltpu.VMEM_SHARED((ACCR, MC), jnp.float32),
    ],
)
def _sc_scatter(dst2_hbm, msg_hbm, zero_hbm, out_hbm,
                didx_v, lidx_v, rows_v, acc_sh):
    c = lax.axis_index("c")
    s = lax.axis_index("s")
    lanes = lax.iota(jnp.int32, 16)

    def do_round(r, _):
        w = 2 * r + c
        nbase = w * NWIN

        pltpu.sync_copy(zero_hbm, acc_sh.at[pl.ds(s * ZROWS, ZROWS)])
        plsc.subcore_barrier()

        def chunk(i, _):
            off = pl.multiple_of(s * PW_S + i * GB, GB)
            row0 = pl.multiple_of(off // 128, 8)
            pltpu.sync_copy(dst2_hbm.at[pl.ds(row0, 8)], didx_v)
            pltpu.sync_copy(msg_hbm.at[pl.ds(off, GB)], rows_v)
            for j in range(8):
                for t in range(8):
                    dvec = didx_v[j, pl.ds(t * 16, 16)]
                    lvec = dvec - nbase
                    gpos = off + (j * 128 + t * 16) + lanes
                    ok = (lvec >= 0) & (lvec < NWIN) & (gpos < E)
                    trash = NWIN + (dvec & 127)
                    lvec = jnp.where(ok, lvec, trash)
                    lidx_v[j, pl.ds(t * 16, 16)] = lvec
            for j in range(8):
                pltpu.sync_copy(rows_v.at[pl.ds(j * 128, 128)],
                                acc_sh.at[lidx_v.at[j]], add=True)
            return 0

        lax.fori_loop(0, NCH_S, chunk, 0)

        plsc.subcore_barrier()
        pltpu.sync_copy(acc_sh.at[pl.ds(s * ZROWS, ZROWS)],
                        out_hbm.at[pl.ds(w * ACCR + s * ZROWS, ZROWS)])
        plsc.subcore_barrier()
        return 0

    lax.fori_loop(0, NROUND, do_round, 0)


def _tc_call(body, grid, in_specs, out_specs, out_shape, args):
    return pl.pallas_call(
        body,
        grid=grid,
        in_specs=in_specs,
        out_specs=out_specs,
        out_shape=out_shape,
    )(*args)


def _row(x):
    return x.reshape(1, -1)


def kernel(power_alloc, beam_alloc, node_power_attn, edge_power_attn,
           edge_index, ptr, batch, params):
    p = params
    f32 = jnp.float32

    pa = power_alloc.reshape(N, 16)
    ba = beam_alloc.reshape(N, 8)
    npa = node_power_attn.reshape(N, -1)
    ea = edge_power_attn.reshape(E, -1)
    ea_p = jnp.concatenate([ea, jnp.zeros((EPAD - E, 16), f32)], axis=0)
    src_p = jnp.concatenate([edge_index[0], jnp.zeros((EPAD - E,), jnp.int32)])
    dst_p = jnp.concatenate([edge_index[1], jnp.zeros((EPAD - E,), jnp.int32)])
    dst2 = dst_p.reshape(EPAD // 128, 128)
    sd = jnp.concatenate([src_p.reshape(EPAD // GBG, 2, 128),
                          dst_p.reshape(EPAD // GBG, 2, 128)],
                         axis=1).reshape(EPAD // GBG * 4, 128)
    batch2 = batch.reshape(N, 1)
    zero_rows = jnp.zeros((ZROWS, MC), f32)

    w_in = p["W_in"]
    idx_pa = [6 * rb + j for rb in range(NUM_RB) for j in range(4)]
    idx_ba = [6 * rb + 4 + j for rb in range(NUM_RB) for j in range(2)]
    wpa = w_in[jnp.array(idx_pa)]
    wba = w_in[jnp.array(idx_ba)]

    nb = N // BN
    spec_n64 = pl.BlockSpec((BN, D), lambda i: (i, 0))
    spec_w = lambda r, c: pl.BlockSpec((r, c), lambda i: (0, 0))

    inp, x = _tc_call(
        _prologue_body, (nb,),
        [pl.BlockSpec((BN, 16), lambda i: (i, 0)),
         pl.BlockSpec((BN, 8), lambda i: (i, 0)),
         pl.BlockSpec((BN, 16), lambda i: (i, 0)),
         spec_w(16, D), spec_w(8, D), spec_w(1, D),
         spec_w(16, D), spec_w(1, D)],
        [spec_n64, spec_n64],
        [jax.ShapeDtypeStruct((N, D), f32)] * 2,
        (pa, ba, npa, wpa, wba, _row(p["b_in"]), p["W_emb"], _row(p["b_emb"])),
    )

    logit = None
    for l in range(L):
        xi, q_t, kv_t = _tc_call(
            _qkv_body, (nb,),
            [spec_n64, spec_n64,
             spec_w(D, D), spec_w(1, D), spec_w(D, D), spec_w(1, D),
             spec_w(D, D), spec_w(1, D)],
            [spec_n64, spec_n64, pl.BlockSpec((BN, 2 * D), lambda i: (i, 0))],
            [jax.ShapeDtypeStruct((N, D), f32),
             jax.ShapeDtypeStruct((N, D), f32),
             jax.ShapeDtypeStruct((N, 2 * D), f32)],
            (x, inp, p["Wq"][l], _row(p["bq"][l]), p["Wk"][l],
             _row(p["bk"][l]), p["Wv"][l], _row(p["bv"][l])),
        )

        kvs, qd = _sc_gather(sd, kv_t, q_t)

        neb = EPAD // BE
        spec_e64 = pl.BlockSpec((BE, D), lambda i: (i, 0))
        msg = _tc_call(
            _edge_body, (neb,),
            [spec_e64, pl.BlockSpec((BE, 2 * D), lambda i: (i, 0)),
             pl.BlockSpec((BE, 16), lambda i: (i, 0)),
             pl.BlockSpec((16, D), lambda i: (0, 0))],
            pl.BlockSpec((BE, MC), lambda i: (i, 0)),
            jax.ShapeDtypeStruct((EPAD, MC), f32),
            (qd, kvs, ea_p, p["We"][l]),
        )

        acc_pad = _sc_scatter(dst2, msg, zero_rows)
        acc = jnp.concatenate(
            [acc_pad[w * ACCR:w * ACCR + NWIN] for w in range(2 * NROUND)],
            axis=0)[:N]

        x, logit = _tc_call(
            _node_body, (nb,),
            [spec_n64, pl.BlockSpec((BN, MC), lambda i: (i, 0)),
             spec_w(D, D), spec_w(1, D), spec_w(1, D), spec_w(1, D),
             spec_w(D, DFF), spec_w(1, DFF), spec_w(DFF, D), spec_w(1, D),
             spec_w(1, D), spec_w(1, D), spec_w(D, NUM_RB), spec_w(1, NUM_RB)],
            [spec_n64, pl.BlockSpec((BN, NUM_RB), lambda i: (i, 0))],
            [jax.ShapeDtypeStruct((N, D), f32),
             jax.ShapeDtypeStruct((N, NUM_RB), f32)],
            (xi, acc, p["Ws"][l], _row(p["bs"][l]), _row(p["g1"][l]),
             _row(p["be1"][l]), p["W1"][l], _row(p["b1"][l]), p["W2"][l],
             _row(p["b2"][l]), _row(p["g2"][l]), _row(p["be2"][l]),
             p["W_actor"], _row(p["b_actor"])),
        )

    sums, cnts, val = _tc_call(
        _pool_body, (nb,),
        [spec_n64, pl.BlockSpec((BN, 1), lambda i: (i, 0)),
         spec_w(D, 1), spec_w(1, 1)],
        [pl.BlockSpec((G, D), lambda i: (0, 0)),
         pl.BlockSpec((G, D), lambda i: (0, 0)),
         pl.BlockSpec((G, 1), lambda i: (0, 0))],
        [jax.ShapeDtypeStruct((G, D), f32), jax.ShapeDtypeStruct((G, D), f32),
         jax.ShapeDtypeStruct((G, 1), f32)],
        (x, batch2, p["W_critic"], p["b_critic"].reshape(1, 1)),
    )
    value = val[:, 0]
    return x, value, logit
